# baseline (speedup 1.0000x reference)
import functools

import jax
import jax.numpy as jnp
import numpy as np
from jax import lax
from jax.experimental import pallas as pl
from jax.experimental.pallas import tpu as pltpu
from jax.experimental.pallas import tpu_sc as plsc

KNN = 32
N = 4096
B = 2
NW = 32
RPW = B * N // NW
CAP = 1024
INF = np.float32(np.inf)
NEG_INF = np.float32(-np.inf)
POS_BIG = np.int32(1 << 30)


def _bf16_round(v):
    bits = lax.bitcast_convert_type(v, jnp.int32)
    r = (bits + 0x7FFF + ((bits >> 16) & 1)) & np.int32(-65536)
    return lax.bitcast_convert_type(r, jnp.float32)


def _sc_body(xt_hbm, c_hbm, idx_hbm, msk_hbm,
             xraw_v, xgb_v, sqm_v, c_v, d2_v,
             cval_v, cidx_v, mf_v, oidx_v, omsk_v):
    cid = lax.axis_index("c")
    sid = lax.axis_index("s")
    wid = sid * 2 + cid
    b = wid // 16
    row0 = (wid % 16) * RPW
    lane = lax.iota(jnp.int32, 16)

    pltpu.sync_copy(xt_hbm.at[b], xraw_v)
    pltpu.sync_copy(c_hbm.at[b], c_v)

    def col_body(t, carry):
        for u in range(4):
            o = t * 64 + u * 16
            xg = []
            for d in range(3):
                s = ((xraw_v[d, pl.ds(o, 16)] + xraw_v[3 + d, pl.ds(o, 16)])
                     + xraw_v[6 + d, pl.ds(o, 16)]
                     + xraw_v[9 + d, pl.ds(o, 16)]) * 0.25
                xg.append(s)
            sq = (xg[0] * xg[0] + xg[1] * xg[1]) + xg[2] * xg[2]
            cc = c_v[pl.ds(o, 16)]
            sqm_v[pl.ds(o, 16)] = jnp.where(cc > 0, sq, INF)
            for d in range(3):
                xgb_v[d, pl.ds(o, 16)] = _bf16_round(xg[d])
        return carry
    lax.fori_loop(0, N // 64, col_body, 0)

    def mf_body(t, p):
        o = t * 16
        mm = c_v[pl.ds(o, 16)] <= 0
        cnt = jnp.max(plsc.all_reduce_population_count(mm))
        plsc.store_compressed(mf_v.at[pl.ds(p, 16)], lane + o, mask=mm)
        return jnp.minimum(p + cnt, np.int32(32))
    lax.fori_loop(0, N // 16, mf_body, np.int32(0))

    def row_body(i, carry):
        ri = row0 + i
        co = (ri // 16) * 16
        onehot = lane == (ri % 16)
        ci = jnp.max(jnp.where(onehot, c_v[pl.ds(co, 16)],
                               np.int32(-2147483647)))

        @pl.when(ci <= 0)
        def _():
            oidx_v[i, pl.ds(0, 16)] = lane
            oidx_v[i, pl.ds(16, 16)] = lane + 16
            omsk_v[i, pl.ds(0, 16)] = jnp.zeros((16,), jnp.float32)
            omsk_v[i, pl.ds(16, 16)] = jnp.zeros((16,), jnp.float32)

        @pl.when(ci > 0)
        def _():
            xi = []
            for d in range(3):
                ch = xgb_v[d, pl.ds(co, 16)]
                xi.append(jnp.sum(jnp.where(onehot, ch, np.float32(0.0))))
            sqi = jnp.sum(jnp.where(onehot, sqm_v[pl.ds(co, 16)],
                                    np.float32(0.0)))

            def grp_body(t, gms):
                gA, gB = gms
                for u in range(8):
                    o = t * 128 + u * 16
                    v = (xi[0] * xgb_v[0, pl.ds(o, 16)]
                         + xi[1] * xgb_v[1, pl.ds(o, 16)]) \
                        + xi[2] * xgb_v[2, pl.ds(o, 16)]
                    d2 = jnp.maximum((sqi + sqm_v[pl.ds(o, 16)]) - 2.0 * v,
                                     0.0)
                    d2_v[pl.ds(o, 16)] = d2
                    if u % 2 == 0:
                        gA = jnp.minimum(gA, d2)
                    else:
                        gB = jnp.minimum(gB, d2)
                return (gA, gB)
            gmA, gmB = lax.fori_loop(
                0, 32, grp_body,
                (jnp.full((16,), INF, jnp.float32),
                 jnp.full((16,), INF, jnp.float32)))
            t0 = jnp.max(jnp.maximum(gmA, gmB))

            def collect(t, pv):
                for u in range(4):
                    o = t * 64 + u * 16
                    d2c = d2_v[pl.ds(o, 16)]
                    mm = (d2c <= t0) & (d2c < INF)
                    cs = plsc.cumsum(jnp.where(mm, np.float32(1.0),
                                               np.float32(0.0)))
                    posn = pv + cs.astype(jnp.int32) - 1
                    plsc.store_scatter(cval_v, [posn], d2c, mask=mm)
                    plsc.store_scatter(cidx_v, [posn], lane + o, mask=mm)
                    cnt = plsc.all_reduce_population_count(mm)
                    pv = jnp.minimum(pv + cnt, np.int32(CAP - 16))
                return pv
            pv = lax.fori_loop(0, N // 64, collect,
                               jnp.zeros((16,), jnp.int32))
            nv = jnp.max(pv)
            nb = (nv + 15) // 16

            def sel_k(k, acc):
                oi0, oi1, om0, om1, fc = acc

                def scan_q(q, mp):
                    mn, pv = mp
                    v = cval_v[pl.ds(q * 16, 16)]
                    lt = v < mn
                    return (jnp.where(lt, v, mn),
                            jnp.where(lt, lane + q * 16, pv))
                mn, pv = lax.fori_loop(
                    0, nb, scan_q,
                    (jnp.full((16,), INF, jnp.float32),
                     jnp.full((16,), POS_BIG, jnp.int32)))
                m = jnp.min(mn)
                pos = jnp.min(jnp.where(mn == m, pv, POS_BIG))
                is_real = m < INF
                pos_s = jnp.where(is_real, pos, 0)
                posv = jnp.zeros((16,), jnp.int32) + pos_s
                jv = plsc.load_gather(cidx_v, [posv])
                fv = plsc.load_gather(mf_v, [fc])
                outj = jnp.where(is_real, jv, fv)
                fc = fc + jnp.where(is_real, 0, 1)
                plsc.store_scatter(cval_v, [posv],
                                   jnp.full((16,), INF, jnp.float32),
                                   mask=(lane == 0) & is_real)
                sel = lane == (k % 16)
                first = k < 16
                mv = jnp.where(is_real, np.float32(1.0), np.float32(0.0))
                mvv = jnp.zeros((16,), jnp.float32) + mv
                oi0 = jnp.where(sel & first, outj, oi0)
                oi1 = jnp.where(sel & (~first), outj, oi1)
                om0 = jnp.where(sel & first, mvv, om0)
                om1 = jnp.where(sel & (~first), mvv, om1)
                return (oi0, oi1, om0, om1, fc)

            z_i = jnp.zeros((16,), jnp.int32)
            z_f = jnp.zeros((16,), jnp.float32)
            oi0, oi1, om0, om1, _ = lax.fori_loop(0, KNN, sel_k,
                                                  (z_i, z_i, z_f, z_f, z_i))
            oidx_v[i, pl.ds(0, 16)] = oi0
            oidx_v[i, pl.ds(16, 16)] = oi1
            omsk_v[i, pl.ds(0, 16)] = om0
            omsk_v[i, pl.ds(16, 16)] = om1

            def clr(q, carry2):
                cval_v[pl.ds(q * 16, 16)] = jnp.full((16,), INF, jnp.float32)
                return carry2
            lax.fori_loop(0, nb, clr, 0)
        return carry

    def prefill(q, carry):
        cval_v[pl.ds(q * 16, 16)] = jnp.full((16,), INF, jnp.float32)
        return carry
    lax.fori_loop(0, CAP // 16, prefill, 0)
    lax.fori_loop(0, RPW, row_body, 0)

    pltpu.sync_copy(oidx_v, idx_hbm.at[pl.ds(wid * RPW, RPW)])
    pltpu.sync_copy(omsk_v, msk_hbm.at[pl.ds(wid * RPW, RPW)])


@jax.jit
def kernel(X, C):
    Xt = X.reshape(B, N, 12).transpose(0, 2, 1)
    Ci = C.astype(jnp.int32)
    mesh = plsc.VectorSubcoreMesh(core_axis_name="c", subcore_axis_name="s")
    run = functools.partial(
        pl.kernel,
        mesh=mesh,
        compiler_params=pltpu.CompilerParams(needs_layout_passes=False,
                                             use_tc_tiling_on_sc=False),
        out_type=[jax.ShapeDtypeStruct((B * N, KNN), jnp.int32),
                  jax.ShapeDtypeStruct((B * N, KNN), jnp.float32)],
        scratch_types=[
            pltpu.VMEM((12, N), jnp.float32),
            pltpu.VMEM((3, N), jnp.float32),
            pltpu.VMEM((N,), jnp.float32),
            pltpu.VMEM((N,), jnp.int32),
            pltpu.VMEM((N,), jnp.float32),
            pltpu.VMEM((CAP,), jnp.float32),
            pltpu.VMEM((CAP,), jnp.int32),
            pltpu.VMEM((48,), jnp.int32),
            pltpu.VMEM((RPW, KNN), jnp.int32),
            pltpu.VMEM((RPW, KNN), jnp.float32),
        ],
    )(_sc_body)
    idx_flat, msk_flat = run(Xt, Ci)
    return idx_flat.reshape(B, N, KNN), msk_flat.reshape(B, N, KNN)


# speedup vs baseline: 1.2211x; 1.2211x over previous
import functools

import jax
import jax.numpy as jnp
import numpy as np
from jax import lax
from jax.experimental import pallas as pl
from jax.experimental.pallas import tpu as pltpu
from jax.experimental.pallas import tpu_sc as plsc

KNN = 32
N = 4096
B = 2
NW = 32
RPW = B * N // NW
CAP = 1024
INF = np.float32(np.inf)
NEG_INF = np.float32(-np.inf)
POS_BIG = np.int32(1 << 30)


def _bf16_round(v):
    bits = lax.bitcast_convert_type(v, jnp.int32)
    r = (bits + 0x7FFF + ((bits >> 16) & 1)) & np.int32(-65536)
    return lax.bitcast_convert_type(r, jnp.float32)


def _sc_body(xt_hbm, c_hbm, idx_hbm, msk_hbm,
             xraw_v, xgb_v, sqm_v, c_v, d2_v,
             cval_v, cidx_v, mf_v, oidx_v, omsk_v):
    cid = lax.axis_index("c")
    sid = lax.axis_index("s")
    wid = sid * 2 + cid
    b = wid // 16
    row0 = (wid % 16) * RPW
    lane = lax.iota(jnp.int32, 16)

    pltpu.sync_copy(xt_hbm.at[b], xraw_v)
    pltpu.sync_copy(c_hbm.at[b], c_v)

    def col_body(t, carry):
        for u in range(4):
            o = t * 64 + u * 16
            xg = []
            for d in range(3):
                s = ((xraw_v[d, pl.ds(o, 16)] + xraw_v[3 + d, pl.ds(o, 16)])
                     + xraw_v[6 + d, pl.ds(o, 16)]
                     + xraw_v[9 + d, pl.ds(o, 16)]) * 0.25
                xg.append(s)
            sq = (xg[0] * xg[0] + xg[1] * xg[1]) + xg[2] * xg[2]
            cc = c_v[pl.ds(o, 16)]
            sqm_v[pl.ds(o, 16)] = jnp.where(cc > 0, sq, INF)
            for d in range(3):
                xgb_v[d, pl.ds(o, 16)] = _bf16_round(xg[d])
        return carry
    lax.fori_loop(0, N // 64, col_body, 0)

    def mf_body(t, p):
        o = t * 16
        mm = c_v[pl.ds(o, 16)] <= 0
        cnt = jnp.max(plsc.all_reduce_population_count(mm))
        plsc.store_compressed(mf_v.at[pl.ds(p, 16)], lane + o, mask=mm)
        return jnp.minimum(p + cnt, np.int32(32))
    lax.fori_loop(0, N // 16, mf_body, np.int32(0))

    def row_body(i, carry):
        ri = row0 + i
        co = (ri // 16) * 16
        onehot = lane == (ri % 16)
        ci = jnp.max(jnp.where(onehot, c_v[pl.ds(co, 16)],
                               np.int32(-2147483647)))

        @pl.when(ci <= 0)
        def _():
            oidx_v[i, pl.ds(0, 16)] = lane
            oidx_v[i, pl.ds(16, 16)] = lane + 16
            omsk_v[i, pl.ds(0, 16)] = jnp.zeros((16,), jnp.float32)
            omsk_v[i, pl.ds(16, 16)] = jnp.zeros((16,), jnp.float32)

        @pl.when(ci > 0)
        def _():
            xi = []
            for d in range(3):
                ch = xgb_v[d, pl.ds(co, 16)]
                xi.append(jnp.sum(jnp.where(onehot, ch, np.float32(0.0))))
            sqi = jnp.sum(jnp.where(onehot, sqm_v[pl.ds(co, 16)],
                                    np.float32(0.0)))

            def grp_body(t, gms):
                gA, gB = gms
                for u in range(8):
                    o = t * 128 + u * 16
                    v = (xi[0] * xgb_v[0, pl.ds(o, 16)]
                         + xi[1] * xgb_v[1, pl.ds(o, 16)]) \
                        + xi[2] * xgb_v[2, pl.ds(o, 16)]
                    d2 = jnp.maximum((sqi + sqm_v[pl.ds(o, 16)]) - 2.0 * v,
                                     0.0)
                    d2_v[pl.ds(o, 16)] = d2
                    if u % 2 == 0:
                        gA = jnp.minimum(gA, d2)
                    else:
                        gB = jnp.minimum(gB, d2)
                return (gA, gB)
            gmA, gmB = lax.fori_loop(
                0, 32, grp_body,
                (jnp.full((16,), INF, jnp.float32),
                 jnp.full((16,), INF, jnp.float32)))
            t0 = jnp.max(jnp.maximum(gmA, gmB))

            def collect(t, p):
                for u in range(4):
                    o = t * 64 + u * 16
                    d2c = d2_v[pl.ds(o, 16)]
                    mm = (d2c <= t0) & (d2c < INF)
                    cnt = jnp.max(plsc.all_reduce_population_count(mm))
                    plsc.store_compressed(cval_v.at[pl.ds(p, 16)], d2c,
                                          mask=mm)
                    plsc.store_compressed(cidx_v.at[pl.ds(p, 16)],
                                          lane + o, mask=mm)
                    p = jnp.minimum(p + cnt, np.int32(CAP - 16))
                return p
            nv = lax.fori_loop(0, N // 64, collect, np.int32(0))
            cval_v[pl.ds(nv, 16)] = jnp.full((16,), INF, jnp.float32)
            nb = (nv + 15) // 16

            def sel_k(k, acc):
                oi0, oi1, om0, om1, fc = acc

                def scan_q(q, mp):
                    mn, pv = mp
                    v = cval_v[pl.ds(q * 16, 16)]
                    lt = v < mn
                    return (jnp.where(lt, v, mn),
                            jnp.where(lt, lane + q * 16, pv))
                mn, pv = lax.fori_loop(
                    0, nb, scan_q,
                    (jnp.full((16,), INF, jnp.float32),
                     jnp.full((16,), POS_BIG, jnp.int32)))
                m = jnp.min(mn)
                pos = jnp.min(jnp.where(mn == m, pv, POS_BIG))
                is_real = m < INF
                pos_s = jnp.where(is_real, pos, 0)
                posv = jnp.zeros((16,), jnp.int32) + pos_s
                jv = plsc.load_gather(cidx_v, [posv])
                fv = plsc.load_gather(mf_v, [fc])
                outj = jnp.where(is_real, jv, fv)
                fc = fc + jnp.where(is_real, 0, 1)
                plsc.store_scatter(cval_v, [posv],
                                   jnp.full((16,), INF, jnp.float32),
                                   mask=(lane == 0) & is_real)
                sel = lane == (k % 16)
                first = k < 16
                mv = jnp.where(is_real, np.float32(1.0), np.float32(0.0))
                mvv = jnp.zeros((16,), jnp.float32) + mv
                oi0 = jnp.where(sel & first, outj, oi0)
                oi1 = jnp.where(sel & (~first), outj, oi1)
                om0 = jnp.where(sel & first, mvv, om0)
                om1 = jnp.where(sel & (~first), mvv, om1)
                return (oi0, oi1, om0, om1, fc)

            z_i = jnp.zeros((16,), jnp.int32)
            z_f = jnp.zeros((16,), jnp.float32)
            oi0, oi1, om0, om1, _ = lax.fori_loop(0, KNN, sel_k,
                                                  (z_i, z_i, z_f, z_f, z_i))
            oidx_v[i, pl.ds(0, 16)] = oi0
            oidx_v[i, pl.ds(16, 16)] = oi1
            omsk_v[i, pl.ds(0, 16)] = om0
            omsk_v[i, pl.ds(16, 16)] = om1

        return carry
    lax.fori_loop(0, RPW, row_body, 0)

    pltpu.sync_copy(oidx_v, idx_hbm.at[pl.ds(wid * RPW, RPW)])
    pltpu.sync_copy(omsk_v, msk_hbm.at[pl.ds(wid * RPW, RPW)])


@jax.jit
def kernel(X, C):
    Xt = X.reshape(B, N, 12).transpose(0, 2, 1)
    Ci = C.astype(jnp.int32)
    mesh = plsc.VectorSubcoreMesh(core_axis_name="c", subcore_axis_name="s")
    run = functools.partial(
        pl.kernel,
        mesh=mesh,
        compiler_params=pltpu.CompilerParams(needs_layout_passes=False,
                                             use_tc_tiling_on_sc=False),
        out_type=[jax.ShapeDtypeStruct((B * N, KNN), jnp.int32),
                  jax.ShapeDtypeStruct((B * N, KNN), jnp.float32)],
        scratch_types=[
            pltpu.VMEM((12, N), jnp.float32),
            pltpu.VMEM((3, N), jnp.float32),
            pltpu.VMEM((N,), jnp.float32),
            pltpu.VMEM((N,), jnp.int32),
            pltpu.VMEM((N,), jnp.float32),
            pltpu.VMEM((CAP,), jnp.float32),
            pltpu.VMEM((CAP,), jnp.int32),
            pltpu.VMEM((48,), jnp.int32),
            pltpu.VMEM((RPW, KNN), jnp.int32),
            pltpu.VMEM((RPW, KNN), jnp.float32),
        ],
    )(_sc_body)
    idx_flat, msk_flat = run(Xt, Ci)
    return idx_flat.reshape(B, N, KNN), msk_flat.reshape(B, N, KNN)


# speedup vs baseline: 1.3619x; 1.1153x over previous
import functools

import jax
import jax.numpy as jnp
import numpy as np
from jax import lax
from jax.experimental import pallas as pl
from jax.experimental.pallas import tpu as pltpu
from jax.experimental.pallas import tpu_sc as plsc

KNN = 32
N = 4096
B = 2
NW = 32
RPW = B * N // NW
CAP = 1024
INF = np.float32(np.inf)
NEG_INF = np.float32(-np.inf)
POS_BIG = np.int32(1 << 30)


def _bf16_round(v):
    bits = lax.bitcast_convert_type(v, jnp.int32)
    r = (bits + 0x7FFF + ((bits >> 16) & 1)) & np.int32(-65536)
    return lax.bitcast_convert_type(r, jnp.float32)


def _sc_body(xt_hbm, c_hbm, idx_hbm, msk_hbm,
             xraw_v, xgb_v, sqm_v, c_v, d2_v,
             cval_v, cidx_v, mf_v, oidx_v, omsk_v):
    cid = lax.axis_index("c")
    sid = lax.axis_index("s")
    wid = sid * 2 + cid
    b = wid // 16
    row0 = (wid % 16) * RPW
    lane = lax.iota(jnp.int32, 16)

    pltpu.sync_copy(xt_hbm.at[b], xraw_v)
    pltpu.sync_copy(c_hbm.at[b], c_v)

    def col_body(t, carry):
        for u in range(4):
            o = t * 64 + u * 16
            xg = []
            for d in range(3):
                s = ((xraw_v[d, pl.ds(o, 16)] + xraw_v[3 + d, pl.ds(o, 16)])
                     + xraw_v[6 + d, pl.ds(o, 16)]
                     + xraw_v[9 + d, pl.ds(o, 16)]) * 0.25
                xg.append(s)
            sq = (xg[0] * xg[0] + xg[1] * xg[1]) + xg[2] * xg[2]
            cc = c_v[pl.ds(o, 16)]
            sqm_v[pl.ds(o, 16)] = jnp.where(cc > 0, sq, INF)
            for d in range(3):
                xgb_v[d, pl.ds(o, 16)] = _bf16_round(xg[d])
        return carry
    lax.fori_loop(0, N // 64, col_body, 0)

    def mf_body(t, p):
        o = t * 16
        mm = c_v[pl.ds(o, 16)] <= 0
        cnt = plsc.all_reduce_population_count(mm)[0]
        plsc.store_compressed(mf_v.at[pl.ds(p, 16)], lane + o, mask=mm)
        return jnp.minimum(p + cnt, np.int32(32))
    lax.fori_loop(0, N // 16, mf_body, np.int32(0))

    def row_body(i, carry):
        ri = row0 + i
        co = (ri // 16) * 16
        onehot = lane == (ri % 16)
        ci = jnp.max(jnp.where(onehot, c_v[pl.ds(co, 16)],
                               np.int32(-2147483647)))

        @pl.when(ci <= 0)
        def _():
            oidx_v[i, pl.ds(0, 16)] = lane
            oidx_v[i, pl.ds(16, 16)] = lane + 16
            omsk_v[i, pl.ds(0, 16)] = jnp.zeros((16,), jnp.float32)
            omsk_v[i, pl.ds(16, 16)] = jnp.zeros((16,), jnp.float32)

        @pl.when(ci > 0)
        def _():
            xi = []
            for d in range(3):
                ch = xgb_v[d, pl.ds(co, 16)]
                xi.append(jnp.sum(jnp.where(onehot, ch, np.float32(0.0))))
            sqi = jnp.sum(jnp.where(onehot, sqm_v[pl.ds(co, 16)],
                                    np.float32(0.0)))

            def grp_body(t, gms):
                gA, gB = gms
                for u in range(8):
                    o = t * 128 + u * 16
                    v = (xi[0] * xgb_v[0, pl.ds(o, 16)]
                         + xi[1] * xgb_v[1, pl.ds(o, 16)]) \
                        + xi[2] * xgb_v[2, pl.ds(o, 16)]
                    d2 = jnp.maximum((sqi + sqm_v[pl.ds(o, 16)]) - 2.0 * v,
                                     0.0)
                    d2_v[pl.ds(o, 16)] = d2
                    if u % 2 == 0:
                        gA = jnp.minimum(gA, d2)
                    else:
                        gB = jnp.minimum(gB, d2)
                return (gA, gB)
            gmA, gmB = lax.fori_loop(
                0, 32, grp_body,
                (jnp.full((16,), INF, jnp.float32),
                 jnp.full((16,), INF, jnp.float32)))
            t0 = jnp.max(jnp.maximum(gmA, gmB))

            def collect(t, p):
                for u in range(4):
                    o = t * 64 + u * 16
                    d2c = d2_v[pl.ds(o, 16)]
                    mm = (d2c <= t0) & (d2c < INF)
                    cnt = plsc.all_reduce_population_count(mm)[0]
                    plsc.store_compressed(cval_v.at[pl.ds(p, 16)], d2c,
                                          mask=mm)
                    plsc.store_compressed(cidx_v.at[pl.ds(p, 16)],
                                          lane + o, mask=mm)
                    p = jnp.minimum(p + cnt, np.int32(CAP - 16))
                return p
            nv = lax.fori_loop(0, N // 64, collect, np.int32(0))
            cval_v[pl.ds(nv, 16)] = jnp.full((16,), INF, jnp.float32)
            nb = (nv + 15) // 16

            def sel_k(k, acc):
                oi0, oi1, om0, om1, fc = acc

                def scan_q(q, mp):
                    mn, pv = mp
                    v = cval_v[pl.ds(q * 16, 16)]
                    lt = v < mn
                    return (jnp.where(lt, v, mn),
                            jnp.where(lt, lane + q * 16, pv))
                mn, pv = lax.fori_loop(
                    0, nb, scan_q,
                    (jnp.full((16,), INF, jnp.float32),
                     jnp.full((16,), POS_BIG, jnp.int32)))
                m = jnp.min(mn)
                pos = jnp.min(jnp.where(mn == m, pv, POS_BIG))
                is_real = m < INF
                pos_s = jnp.where(is_real, pos, 0)
                posv = jnp.zeros((16,), jnp.int32) + pos_s
                jv = plsc.load_gather(cidx_v, [posv])
                fv = plsc.load_gather(mf_v, [fc])
                outj = jnp.where(is_real, jv, fv)
                fc = fc + jnp.where(is_real, 0, 1)
                plsc.store_scatter(cval_v, [posv],
                                   jnp.full((16,), INF, jnp.float32),
                                   mask=(lane == 0) & is_real)
                sel = lane == (k % 16)
                first = k < 16
                mv = jnp.where(is_real, np.float32(1.0), np.float32(0.0))
                mvv = jnp.zeros((16,), jnp.float32) + mv
                oi0 = jnp.where(sel & first, outj, oi0)
                oi1 = jnp.where(sel & (~first), outj, oi1)
                om0 = jnp.where(sel & first, mvv, om0)
                om1 = jnp.where(sel & (~first), mvv, om1)
                return (oi0, oi1, om0, om1, fc)

            z_i = jnp.zeros((16,), jnp.int32)
            z_f = jnp.zeros((16,), jnp.float32)
            oi0, oi1, om0, om1, _ = lax.fori_loop(0, KNN, sel_k,
                                                  (z_i, z_i, z_f, z_f, z_i))
            oidx_v[i, pl.ds(0, 16)] = oi0
            oidx_v[i, pl.ds(16, 16)] = oi1
            omsk_v[i, pl.ds(0, 16)] = om0
            omsk_v[i, pl.ds(16, 16)] = om1

        return carry
    lax.fori_loop(0, RPW, row_body, 0)

    pltpu.sync_copy(oidx_v, idx_hbm.at[pl.ds(wid * RPW, RPW)])
    pltpu.sync_copy(omsk_v, msk_hbm.at[pl.ds(wid * RPW, RPW)])


@jax.jit
def kernel(X, C):
    Xt = X.reshape(B, N, 12).transpose(0, 2, 1)
    Ci = C.astype(jnp.int32)
    mesh = plsc.VectorSubcoreMesh(core_axis_name="c", subcore_axis_name="s")
    run = functools.partial(
        pl.kernel,
        mesh=mesh,
        compiler_params=pltpu.CompilerParams(needs_layout_passes=False,
                                             use_tc_tiling_on_sc=False),
        out_type=[jax.ShapeDtypeStruct((B * N, KNN), jnp.int32),
                  jax.ShapeDtypeStruct((B * N, KNN), jnp.float32)],
        scratch_types=[
            pltpu.VMEM((12, N), jnp.float32),
            pltpu.VMEM((3, N), jnp.float32),
            pltpu.VMEM((N,), jnp.float32),
            pltpu.VMEM((N,), jnp.int32),
            pltpu.VMEM((N,), jnp.float32),
            pltpu.VMEM((CAP,), jnp.float32),
            pltpu.VMEM((CAP,), jnp.int32),
            pltpu.VMEM((48,), jnp.int32),
            pltpu.VMEM((RPW, KNN), jnp.int32),
            pltpu.VMEM((RPW, KNN), jnp.float32),
        ],
    )(_sc_body)
    idx_flat, msk_flat = run(Xt, Ci)
    return idx_flat.reshape(B, N, KNN), msk_flat.reshape(B, N, KNN)


# speedup vs baseline: 1.9987x; 1.4675x over previous
import functools

import jax
import jax.numpy as jnp
import numpy as np
from jax import lax
from jax.experimental import pallas as pl
from jax.experimental.pallas import tpu as pltpu
from jax.experimental.pallas import tpu_sc as plsc

KNN = 32
N = 4096
B = 2
NW = 32
RPW = B * N // NW
CAP = 1024
INF = np.float32(np.inf)
POS_BIG = np.int32(1 << 30)


def _bf16_round(v):
    bits = lax.bitcast_convert_type(v, jnp.int32)
    r = (bits + 0x7FFF + ((bits >> 16) & 1)) & np.int32(-65536)
    return lax.bitcast_convert_type(r, jnp.float32)


def _sc_body(xt_hbm, c_hbm, idx_hbm, msk_hbm,
             xraw_v, xgb_v, sqm_v, c_v, d2a_v, d2b_v,
             cvala_v, cidxa_v, cvalb_v, cidxb_v, mf_v, oidx_v, omsk_v):
    cid = lax.axis_index("c")
    sid = lax.axis_index("s")
    wid = sid * 2 + cid
    b = wid // 16
    row0 = (wid % 16) * RPW
    lane = lax.iota(jnp.int32, 16)

    pltpu.sync_copy(xt_hbm.at[b], xraw_v)
    pltpu.sync_copy(c_hbm.at[b], c_v)

    def col_body(t, carry):
        for u in range(4):
            o = t * 64 + u * 16
            xg = []
            for d in range(3):
                s = ((xraw_v[d, pl.ds(o, 16)] + xraw_v[3 + d, pl.ds(o, 16)])
                     + xraw_v[6 + d, pl.ds(o, 16)]
                     + xraw_v[9 + d, pl.ds(o, 16)]) * 0.25
                xg.append(s)
            sq = (xg[0] * xg[0] + xg[1] * xg[1]) + xg[2] * xg[2]
            cc = c_v[pl.ds(o, 16)]
            sqm_v[pl.ds(o, 16)] = jnp.where(cc > 0, sq, INF)
            for d in range(3):
                xgb_v[d, pl.ds(o, 16)] = _bf16_round(xg[d])
        return carry
    lax.fori_loop(0, N // 64, col_body, 0)

    def mf_body(t, p):
        o = t * 16
        mm = c_v[pl.ds(o, 16)] <= 0
        cnt = plsc.all_reduce_population_count(mm)[0]
        plsc.store_compressed(mf_v.at[pl.ds(p, 16)], lane + o, mask=mm)
        return jnp.minimum(p + cnt, np.int32(32))
    lax.fori_loop(0, N // 16, mf_body, np.int32(0))

    def pair_body(ip, carry):
        ria = row0 + ip * 2
        rib = ria + 1
        coa = (ria // 16) * 16
        oha = lane == (ria % 16)
        ohb = lane == (rib % 16)
        cca = c_v[pl.ds(coa, 16)]
        cia = jnp.max(jnp.where(oha, cca, np.int32(-2147483647)))
        cib = jnp.max(jnp.where(ohb, cca, np.int32(-2147483647)))

        xia, xib = [], []
        for d in range(3):
            ch = xgb_v[d, pl.ds(coa, 16)]
            xia.append(jnp.sum(jnp.where(oha, ch, np.float32(0.0))))
            xib.append(jnp.sum(jnp.where(ohb, ch, np.float32(0.0))))
        sqch = sqm_v[pl.ds(coa, 16)]
        sqia = jnp.sum(jnp.where(oha & (cca > 0), sqch, np.float32(0.0)))
        sqib = jnp.sum(jnp.where(ohb & (cca > 0), sqch, np.float32(0.0)))
        sqia = jnp.where(cia > 0, sqia, INF)
        sqib = jnp.where(cib > 0, sqib, INF)

        def grp_body(t, gms):
            gAa, gBa, gAb, gBb = gms
            for u in range(8):
                o = t * 128 + u * 16
                xj0 = xgb_v[0, pl.ds(o, 16)]
                xj1 = xgb_v[1, pl.ds(o, 16)]
                xj2 = xgb_v[2, pl.ds(o, 16)]
                sqv = sqm_v[pl.ds(o, 16)]
                va = (xia[0] * xj0 + xia[1] * xj1) + xia[2] * xj2
                vb = (xib[0] * xj0 + xib[1] * xj1) + xib[2] * xj2
                d2a = jnp.maximum((sqia + sqv) - 2.0 * va, 0.0)
                d2b = jnp.maximum((sqib + sqv) - 2.0 * vb, 0.0)
                d2a_v[pl.ds(o, 16)] = d2a
                d2b_v[pl.ds(o, 16)] = d2b
                if u % 2 == 0:
                    gAa = jnp.minimum(gAa, d2a)
                    gAb = jnp.minimum(gAb, d2b)
                else:
                    gBa = jnp.minimum(gBa, d2a)
                    gBb = jnp.minimum(gBb, d2b)
            return (gAa, gBa, gAb, gBb)
        inf16 = jnp.full((16,), INF, jnp.float32)
        gAa, gBa, gAb, gBb = lax.fori_loop(0, 32, grp_body,
                                           (inf16, inf16, inf16, inf16))
        t0a = jnp.max(jnp.maximum(gAa, gBa))
        t0b = jnp.max(jnp.maximum(gAb, gBb))

        def collect(t, ps):
            pa, pb = ps
            for u in range(4):
                o = t * 64 + u * 16
                d2ca = d2a_v[pl.ds(o, 16)]
                d2cb = d2b_v[pl.ds(o, 16)]
                mma = (d2ca <= t0a) & (d2ca < INF)
                mmb = (d2cb <= t0b) & (d2cb < INF)
                cnta = plsc.all_reduce_population_count(mma)[0]
                cntb = plsc.all_reduce_population_count(mmb)[0]
                plsc.store_compressed(cvala_v.at[pl.ds(pa, 16)], d2ca,
                                      mask=mma)
                plsc.store_compressed(cidxa_v.at[pl.ds(pa, 16)], lane + o,
                                      mask=mma)
                plsc.store_compressed(cvalb_v.at[pl.ds(pb, 16)], d2cb,
                                      mask=mmb)
                plsc.store_compressed(cidxb_v.at[pl.ds(pb, 16)], lane + o,
                                      mask=mmb)
                pa = jnp.minimum(pa + cnta, np.int32(CAP - 16))
                pb = jnp.minimum(pb + cntb, np.int32(CAP - 16))
            return (pa, pb)
        nva, nvb = lax.fori_loop(0, N // 64, collect,
                                 (np.int32(0), np.int32(0)))
        nb = jnp.maximum((nva + 15) // 16, (nvb + 15) // 16)

        def sel_k(k, acc):
            (oia0, oia1, oma0, oma1, fca,
             oib0, oib1, omb0, omb1, fcb) = acc

            def scan_q(q, mp):
                mna, pva, mnb, pvb = mp
                va = cvala_v[pl.ds(q * 16, 16)]
                vb = cvalb_v[pl.ds(q * 16, 16)]
                pq = lane + q * 16
                lta = va < mna
                ltb = vb < mnb
                return (jnp.where(lta, va, mna), jnp.where(lta, pq, pva),
                        jnp.where(ltb, vb, mnb), jnp.where(ltb, pq, pvb))
            big16 = jnp.full((16,), POS_BIG, jnp.int32)
            mna, pva, mnb, pvb = lax.fori_loop(
                0, nb, scan_q, (inf16, big16, inf16, big16))
            ma = jnp.min(mna)
            mb = jnp.min(mnb)
            posa = jnp.min(jnp.where(mna == ma, pva, POS_BIG))
            posb = jnp.min(jnp.where(mnb == mb, pvb, POS_BIG))
            reala = ma < INF
            realb = mb < INF
            posva = jnp.zeros((16,), jnp.int32) + jnp.where(reala, posa, 0)
            posvb = jnp.zeros((16,), jnp.int32) + jnp.where(realb, posb, 0)
            jva = plsc.load_gather(cidxa_v, [posva])
            jvb = plsc.load_gather(cidxb_v, [posvb])
            fva = plsc.load_gather(mf_v, [fca])
            fvb = plsc.load_gather(mf_v, [fcb])
            outja = jnp.where(reala, jva, fva)
            outjb = jnp.where(realb, jvb, fvb)
            fca = fca + jnp.where(reala, 0, 1)
            fcb = fcb + jnp.where(realb, 0, 1)
            plsc.store_scatter(cvala_v, [posva], inf16,
                               mask=(lane == 0) & reala)
            plsc.store_scatter(cvalb_v, [posvb], inf16,
                               mask=(lane == 0) & realb)
            sel = lane == (k % 16)
            first = k < 16
            mva = jnp.zeros((16,), jnp.float32) + jnp.where(
                reala, np.float32(1.0), np.float32(0.0))
            mvb = jnp.zeros((16,), jnp.float32) + jnp.where(
                realb, np.float32(1.0), np.float32(0.0))
            oia0 = jnp.where(sel & first, outja, oia0)
            oia1 = jnp.where(sel & (~first), outja, oia1)
            oma0 = jnp.where(sel & first, mva, oma0)
            oma1 = jnp.where(sel & (~first), mva, oma1)
            oib0 = jnp.where(sel & first, outjb, oib0)
            oib1 = jnp.where(sel & (~first), outjb, oib1)
            omb0 = jnp.where(sel & first, mvb, omb0)
            omb1 = jnp.where(sel & (~first), mvb, omb1)
            return (oia0, oia1, oma0, oma1, fca,
                    oib0, oib1, omb0, omb1, fcb)

        z_i = jnp.zeros((16,), jnp.int32)
        z_f = jnp.zeros((16,), jnp.float32)
        (oia0, oia1, oma0, oma1, _,
         oib0, oib1, omb0, omb1, _) = lax.fori_loop(
            0, KNN, sel_k, (z_i, z_i, z_f, z_f, z_i,
                            z_i, z_i, z_f, z_f, z_i))

        mra = cia <= 0
        mrb = cib <= 0
        ia = ip * 2
        ib = ia + 1
        oidx_v[ia, pl.ds(0, 16)] = jnp.where(mra, lane, oia0)
        oidx_v[ia, pl.ds(16, 16)] = jnp.where(mra, lane + 16, oia1)
        omsk_v[ia, pl.ds(0, 16)] = jnp.where(mra, np.float32(0.0), oma0)
        omsk_v[ia, pl.ds(16, 16)] = jnp.where(mra, np.float32(0.0), oma1)
        oidx_v[ib, pl.ds(0, 16)] = jnp.where(mrb, lane, oib0)
        oidx_v[ib, pl.ds(16, 16)] = jnp.where(mrb, lane + 16, oib1)
        omsk_v[ib, pl.ds(0, 16)] = jnp.where(mrb, np.float32(0.0), omb0)
        omsk_v[ib, pl.ds(16, 16)] = jnp.where(mrb, np.float32(0.0), omb1)

        def clr(q, c2):
            cvala_v[pl.ds(q * 16, 16)] = inf16
            cvalb_v[pl.ds(q * 16, 16)] = inf16
            return c2
        lax.fori_loop(0, nb, clr, 0)
        return carry

    def prefill(q, carry):
        cvala_v[pl.ds(q * 16, 16)] = jnp.full((16,), INF, jnp.float32)
        cvalb_v[pl.ds(q * 16, 16)] = jnp.full((16,), INF, jnp.float32)
        return carry
    lax.fori_loop(0, CAP // 16, prefill, 0)
    lax.fori_loop(0, RPW // 2, pair_body, 0)

    pltpu.sync_copy(oidx_v, idx_hbm.at[pl.ds(wid * RPW, RPW)])
    pltpu.sync_copy(omsk_v, msk_hbm.at[pl.ds(wid * RPW, RPW)])


@jax.jit
def kernel(X, C):
    Xt = X.reshape(B, N, 12).transpose(0, 2, 1)
    Ci = C.astype(jnp.int32)
    mesh = plsc.VectorSubcoreMesh(core_axis_name="c", subcore_axis_name="s")
    run = functools.partial(
        pl.kernel,
        mesh=mesh,
        compiler_params=pltpu.CompilerParams(needs_layout_passes=False,
                                             use_tc_tiling_on_sc=False),
        out_type=[jax.ShapeDtypeStruct((B * N, KNN), jnp.int32),
                  jax.ShapeDtypeStruct((B * N, KNN), jnp.float32)],
        scratch_types=[
            pltpu.VMEM((12, N), jnp.float32),
            pltpu.VMEM((3, N), jnp.float32),
            pltpu.VMEM((N,), jnp.float32),
            pltpu.VMEM((N,), jnp.int32),
            pltpu.VMEM((N,), jnp.float32),
            pltpu.VMEM((N,), jnp.float32),
            pltpu.VMEM((CAP,), jnp.float32),
            pltpu.VMEM((CAP,), jnp.int32),
            pltpu.VMEM((CAP,), jnp.float32),
            pltpu.VMEM((CAP,), jnp.int32),
            pltpu.VMEM((48,), jnp.int32),
            pltpu.VMEM((RPW, KNN), jnp.int32),
            pltpu.VMEM((RPW, KNN), jnp.float32),
        ],
    )(_sc_body)
    idx_flat, msk_flat = run(Xt, Ci)
    return idx_flat.reshape(B, N, KNN), msk_flat.reshape(B, N, KNN)


# speedup vs baseline: 2.2271x; 1.1143x over previous
import functools

import jax
import jax.numpy as jnp
import numpy as np
from jax import lax
from jax.experimental import pallas as pl
from jax.experimental.pallas import tpu as pltpu
from jax.experimental.pallas import tpu_sc as plsc

KNN = 32
N = 4096
B = 2
NW = 32
RPW = B * N // NW
CAP = 1024
INF = np.float32(np.inf)
POS_BIG = np.int32(1 << 30)


def _bf16_round(v):
    bits = lax.bitcast_convert_type(v, jnp.int32)
    r = (bits + 0x7FFF + ((bits >> 16) & 1)) & np.int32(-65536)
    return lax.bitcast_convert_type(r, jnp.float32)


def _sc_body(xt_hbm, c_hbm, idx_hbm, msk_hbm,
             xraw_v, xgb_v, sqm_v, c_v, d2a_v, d2b_v,
             cvala_v, cidxa_v, cvalb_v, cidxb_v, mf_v, oidx_v, omsk_v):
    cid = lax.axis_index("c")
    sid = lax.axis_index("s")
    wid = sid * 2 + cid
    b = wid // 16
    row0 = (wid % 16) * RPW
    lane = lax.iota(jnp.int32, 16)

    pltpu.sync_copy(xt_hbm.at[b], xraw_v)
    pltpu.sync_copy(c_hbm.at[b], c_v)

    def col_body(t, carry):
        for u in range(4):
            o = t * 64 + u * 16
            xg = []
            for d in range(3):
                s = ((xraw_v[d, pl.ds(o, 16)] + xraw_v[3 + d, pl.ds(o, 16)])
                     + xraw_v[6 + d, pl.ds(o, 16)]
                     + xraw_v[9 + d, pl.ds(o, 16)]) * 0.25
                xg.append(s)
            sq = (xg[0] * xg[0] + xg[1] * xg[1]) + xg[2] * xg[2]
            cc = c_v[pl.ds(o, 16)]
            sqm_v[pl.ds(o, 16)] = jnp.where(cc > 0, sq, INF)
            for d in range(3):
                xgb_v[d, pl.ds(o, 16)] = _bf16_round(xg[d])
        return carry
    lax.fori_loop(0, N // 64, col_body, 0)

    def mf_body(t, p):
        o = t * 16
        mm = c_v[pl.ds(o, 16)] <= 0
        cnt = plsc.all_reduce_population_count(mm)[0]
        plsc.store_compressed(mf_v.at[pl.ds(p, 16)], lane + o, mask=mm)
        return jnp.minimum(p + cnt, np.int32(32))
    lax.fori_loop(0, N // 16, mf_body, np.int32(0))

    def pair_body(ip, carry):
        ria = row0 + ip * 2
        rib = ria + 1
        coa = (ria // 16) * 16
        oha = lane == (ria % 16)
        ohb = lane == (rib % 16)
        cca = c_v[pl.ds(coa, 16)]
        cia = jnp.max(jnp.where(oha, cca, np.int32(-2147483647)))
        cib = jnp.max(jnp.where(ohb, cca, np.int32(-2147483647)))

        xia, xib = [], []
        for d in range(3):
            ch = xgb_v[d, pl.ds(coa, 16)]
            xia.append(jnp.sum(jnp.where(oha, ch, np.float32(0.0))))
            xib.append(jnp.sum(jnp.where(ohb, ch, np.float32(0.0))))
        sqch = sqm_v[pl.ds(coa, 16)]
        sqia = jnp.sum(jnp.where(oha & (cca > 0), sqch, np.float32(0.0)))
        sqib = jnp.sum(jnp.where(ohb & (cca > 0), sqch, np.float32(0.0)))
        sqia = jnp.where(cia > 0, sqia, INF)
        sqib = jnp.where(cib > 0, sqib, INF)

        def grp_body(t, gms):
            gAa, gBa, gAb, gBb = gms
            for u in range(8):
                o = t * 128 + u * 16
                xj0 = xgb_v[0, pl.ds(o, 16)]
                xj1 = xgb_v[1, pl.ds(o, 16)]
                xj2 = xgb_v[2, pl.ds(o, 16)]
                sqv = sqm_v[pl.ds(o, 16)]
                va = (xia[0] * xj0 + xia[1] * xj1) + xia[2] * xj2
                vb = (xib[0] * xj0 + xib[1] * xj1) + xib[2] * xj2
                d2a = jnp.maximum((sqia + sqv) - 2.0 * va, 0.0)
                d2b = jnp.maximum((sqib + sqv) - 2.0 * vb, 0.0)
                d2a_v[pl.ds(o, 16)] = d2a
                d2b_v[pl.ds(o, 16)] = d2b
                if u % 2 == 0:
                    gAa = jnp.minimum(gAa, d2a)
                    gAb = jnp.minimum(gAb, d2b)
                else:
                    gBa = jnp.minimum(gBa, d2a)
                    gBb = jnp.minimum(gBb, d2b)
            return (gAa, gBa, gAb, gBb)
        inf16 = jnp.full((16,), INF, jnp.float32)
        gAa, gBa, gAb, gBb = lax.fori_loop(0, 32, grp_body,
                                           (inf16, inf16, inf16, inf16))
        t0a = jnp.max(jnp.maximum(gAa, gBa))
        t0b = jnp.max(jnp.maximum(gAb, gBb))

        def collect(t, ps):
            pa, pb = ps
            for u in range(4):
                o = t * 64 + u * 16
                d2ca = d2a_v[pl.ds(o, 16)]
                d2cb = d2b_v[pl.ds(o, 16)]
                mma = (d2ca <= t0a) & (d2ca < INF)
                mmb = (d2cb <= t0b) & (d2cb < INF)
                cnta = plsc.all_reduce_population_count(mma)[0]
                cntb = plsc.all_reduce_population_count(mmb)[0]
                plsc.store_compressed(cvala_v.at[pl.ds(pa, 16)], d2ca,
                                      mask=mma)
                plsc.store_compressed(cidxa_v.at[pl.ds(pa, 16)], lane + o,
                                      mask=mma)
                plsc.store_compressed(cvalb_v.at[pl.ds(pb, 16)], d2cb,
                                      mask=mmb)
                plsc.store_compressed(cidxb_v.at[pl.ds(pb, 16)], lane + o,
                                      mask=mmb)
                pa = jnp.minimum(pa + cnta, np.int32(CAP - 16))
                pb = jnp.minimum(pb + cntb, np.int32(CAP - 16))
            return (pa, pb)
        nva, nvb = lax.fori_loop(0, N // 64, collect,
                                 (np.int32(0), np.int32(0)))
        nb = jnp.maximum((nva + 15) // 16, (nvb + 15) // 16)

        def sel_k(k, acc):
            (oia0, oia1, oma0, oma1, fca,
             oib0, oib1, omb0, omb1, fcb) = acc

            def scan_q(q2, mp):
                mna, pva, mnb, pvb = mp
                for w in range(2):
                    q = q2 * 2 + w
                    va = cvala_v[pl.ds(q * 16, 16)]
                    vb = cvalb_v[pl.ds(q * 16, 16)]
                    pq = lane + q * 16
                    lta = va < mna
                    ltb = vb < mnb
                    mna = jnp.where(lta, va, mna)
                    pva = jnp.where(lta, pq, pva)
                    mnb = jnp.where(ltb, vb, mnb)
                    pvb = jnp.where(ltb, pq, pvb)
                return (mna, pva, mnb, pvb)
            big16 = jnp.full((16,), POS_BIG, jnp.int32)
            mna, pva, mnb, pvb = lax.fori_loop(
                0, (nb + 1) // 2, scan_q, (inf16, big16, inf16, big16))
            ma = jnp.min(mna)
            mb = jnp.min(mnb)
            posa = jnp.min(jnp.where(mna == ma, pva, POS_BIG))
            posb = jnp.min(jnp.where(mnb == mb, pvb, POS_BIG))
            reala = ma < INF
            realb = mb < INF
            posva = jnp.zeros((16,), jnp.int32) + jnp.where(reala, posa, 0)
            posvb = jnp.zeros((16,), jnp.int32) + jnp.where(realb, posb, 0)
            jva = plsc.load_gather(cidxa_v, [posva])
            jvb = plsc.load_gather(cidxb_v, [posvb])
            fva = plsc.load_gather(mf_v, [fca])
            fvb = plsc.load_gather(mf_v, [fcb])
            outja = jnp.where(reala, jva, fva)
            outjb = jnp.where(realb, jvb, fvb)
            fca = fca + jnp.where(reala, 0, 1)
            fcb = fcb + jnp.where(realb, 0, 1)
            plsc.store_scatter(cvala_v, [posva], inf16,
                               mask=(lane == 0) & reala)
            plsc.store_scatter(cvalb_v, [posvb], inf16,
                               mask=(lane == 0) & realb)
            sel = lane == (k % 16)
            first = k < 16
            mva = jnp.zeros((16,), jnp.float32) + jnp.where(
                reala, np.float32(1.0), np.float32(0.0))
            mvb = jnp.zeros((16,), jnp.float32) + jnp.where(
                realb, np.float32(1.0), np.float32(0.0))
            oia0 = jnp.where(sel & first, outja, oia0)
            oia1 = jnp.where(sel & (~first), outja, oia1)
            oma0 = jnp.where(sel & first, mva, oma0)
            oma1 = jnp.where(sel & (~first), mva, oma1)
            oib0 = jnp.where(sel & first, outjb, oib0)
            oib1 = jnp.where(sel & (~first), outjb, oib1)
            omb0 = jnp.where(sel & first, mvb, omb0)
            omb1 = jnp.where(sel & (~first), mvb, omb1)
            return (oia0, oia1, oma0, oma1, fca,
                    oib0, oib1, omb0, omb1, fcb)

        z_i = jnp.zeros((16,), jnp.int32)
        z_f = jnp.zeros((16,), jnp.float32)
        (oia0, oia1, oma0, oma1, _,
         oib0, oib1, omb0, omb1, _) = lax.fori_loop(
            0, KNN, sel_k, (z_i, z_i, z_f, z_f, z_i,
                            z_i, z_i, z_f, z_f, z_i))

        mra = cia <= 0
        mrb = cib <= 0
        ia = ip * 2
        ib = ia + 1
        oidx_v[ia, pl.ds(0, 16)] = jnp.where(mra, lane, oia0)
        oidx_v[ia, pl.ds(16, 16)] = jnp.where(mra, lane + 16, oia1)
        omsk_v[ia, pl.ds(0, 16)] = jnp.where(mra, np.float32(0.0), oma0)
        omsk_v[ia, pl.ds(16, 16)] = jnp.where(mra, np.float32(0.0), oma1)
        oidx_v[ib, pl.ds(0, 16)] = jnp.where(mrb, lane, oib0)
        oidx_v[ib, pl.ds(16, 16)] = jnp.where(mrb, lane + 16, oib1)
        omsk_v[ib, pl.ds(0, 16)] = jnp.where(mrb, np.float32(0.0), omb0)
        omsk_v[ib, pl.ds(16, 16)] = jnp.where(mrb, np.float32(0.0), omb1)

        def clr(q, c2):
            cvala_v[pl.ds(q * 16, 16)] = inf16
            cvalb_v[pl.ds(q * 16, 16)] = inf16
            return c2
        lax.fori_loop(0, nb, clr, 0)
        return carry

    def prefill(q, carry):
        cvala_v[pl.ds(q * 16, 16)] = jnp.full((16,), INF, jnp.float32)
        cvalb_v[pl.ds(q * 16, 16)] = jnp.full((16,), INF, jnp.float32)
        return carry
    lax.fori_loop(0, CAP // 16, prefill, 0)
    lax.fori_loop(0, RPW // 2, pair_body, 0)

    pltpu.sync_copy(oidx_v, idx_hbm.at[pl.ds(wid * RPW, RPW)])
    pltpu.sync_copy(omsk_v, msk_hbm.at[pl.ds(wid * RPW, RPW)])


@jax.jit
def kernel(X, C):
    Xt = X.reshape(B, N, 12).transpose(0, 2, 1)
    Ci = C.astype(jnp.int32)
    mesh = plsc.VectorSubcoreMesh(core_axis_name="c", subcore_axis_name="s")
    run = functools.partial(
        pl.kernel,
        mesh=mesh,
        compiler_params=pltpu.CompilerParams(needs_layout_passes=False,
                                             use_tc_tiling_on_sc=False),
        out_type=[jax.ShapeDtypeStruct((B * N, KNN), jnp.int32),
                  jax.ShapeDtypeStruct((B * N, KNN), jnp.float32)],
        scratch_types=[
            pltpu.VMEM((12, N), jnp.float32),
            pltpu.VMEM((3, N), jnp.float32),
            pltpu.VMEM((N,), jnp.float32),
            pltpu.VMEM((N,), jnp.int32),
            pltpu.VMEM((N,), jnp.float32),
            pltpu.VMEM((N,), jnp.float32),
            pltpu.VMEM((CAP,), jnp.float32),
            pltpu.VMEM((CAP,), jnp.int32),
            pltpu.VMEM((CAP,), jnp.float32),
            pltpu.VMEM((CAP,), jnp.int32),
            pltpu.VMEM((48,), jnp.int32),
            pltpu.VMEM((RPW, KNN), jnp.int32),
            pltpu.VMEM((RPW, KNN), jnp.float32),
        ],
    )(_sc_body)
    idx_flat, msk_flat = run(Xt, Ci)
    return idx_flat.reshape(B, N, KNN), msk_flat.reshape(B, N, KNN)
